# baseline (speedup 1.0000x reference)
import functools

import jax
import jax.numpy as jnp
from jax import lax
from jax.experimental import pallas as pl
from jax.experimental.pallas import tpu as pltpu
from jax.experimental.pallas import tpu_sc as plsc

D = 128
NUM_CORES = 2
NUM_SUBCORES = 16
NW = NUM_CORES * NUM_SUBCORES
G = 4


@jax.jit
def _lookup(idx, embeddings):
    nb, row = idx.shape
    per_tile = nb // NW
    n_groups = per_tile // G
    assert per_tile % G == 0 and n_groups % 2 == 0 and n_groups >= 4
    mesh = plsc.VectorSubcoreMesh(core_axis_name="c", subcore_axis_name="s")

    @functools.partial(
        pl.kernel,
        mesh=mesh,
        out_type=jax.ShapeDtypeStruct((nb, row, D), jnp.float32),
        compiler_params=pltpu.CompilerParams(use_tc_tiling_on_sc=True),
        scratch_types=[
            pltpu.VMEM((per_tile, row), jnp.int32),
            pltpu.VMEM((G, row, D), jnp.float32),
            pltpu.VMEM((G, row, D), jnp.float32),
            pltpu.SemaphoreType.DMA,
            pltpu.SemaphoreType.DMA,
        ],
    )
    def k(idx_hbm, table_hbm, out_hbm, idx_v, buf0, buf1, sem0, sem1):
        wid = lax.axis_index("s") * NUM_CORES + lax.axis_index("c")
        b0 = wid * per_tile
        pltpu.sync_copy(idx_hbm.at[pl.ds(b0, per_tile)], idx_v)

        bufs = (buf0, buf1)
        sems = (sem0, sem1)

        def streams(g, buf, sem):
            return [
                pltpu.make_async_copy(
                    table_hbm.at[idx_v.at[g * G + i]], buf.at[i], sem
                )
                for i in range(G)
            ]

        def fire(g, buf, sem):
            for st in streams(g, buf, sem):
                st.start()

        def drain(g, buf, sem):
            for st in streams(g, buf, sem):
                st.wait()

        def flush(g, buf):
            pltpu.sync_copy(buf, out_hbm.at[pl.ds(b0 + g * G, G)])

        fire(0, buf0, sem0)

        def body(h, carry):
            for j in range(2):
                g = 2 * h + j
                buf, sem = bufs[j], sems[j]
                drain(g, buf, sem)
                fire(g + 1, bufs[1 - j], sems[1 - j])
                flush(g, buf)
            return carry

        lax.fori_loop(0, n_groups // 2 - 1, body, 0)
        g = n_groups - 2
        drain(g, buf0, sem0)
        fire(g + 1, buf1, sem1)
        flush(g, buf0)
        drain(g + 1, buf1, sem1)
        flush(g + 1, buf1)

    return k(idx, embeddings)


def kernel(inputs, embeddings):
    return _lookup(inputs.astype(jnp.int32), embeddings)


# speedup vs baseline: 1.5018x; 1.5018x over previous
import functools

import jax
import jax.numpy as jnp
from jax import lax
from jax.experimental import pallas as pl
from jax.experimental.pallas import tpu as pltpu
from jax.experimental.pallas import tpu_sc as plsc

D = 128
NUM_CORES = 2
NUM_SUBCORES = 16
NW = NUM_CORES * NUM_SUBCORES


@jax.jit
def _lookup_t(idx_t, embeddings):
    row, nb = idx_t.shape
    per_tile = nb // NW
    assert nb % NW == 0 and row % 2 == 0
    mesh = plsc.VectorSubcoreMesh(core_axis_name="c", subcore_axis_name="s")

    @functools.partial(
        pl.kernel,
        mesh=mesh,
        out_type=jax.ShapeDtypeStruct((row, nb, D), jnp.float32),
        scratch_types=[
            pltpu.VMEM((row, per_tile), jnp.int32),
            pltpu.VMEM((per_tile, D), jnp.float32),
            pltpu.VMEM((per_tile, D), jnp.float32),
            pltpu.SemaphoreType.DMA,
            pltpu.SemaphoreType.DMA,
        ],
    )
    def k(idx_hbm, table_hbm, out_hbm, idx_v, buf0, buf1, sem0, sem1):
        wid = lax.axis_index("s") * NUM_CORES + lax.axis_index("c")
        b0 = wid * per_tile
        pltpu.sync_copy(idx_hbm.at[:, pl.ds(b0, per_tile)], idx_v)

        bufs = (buf0, buf1)
        sems = (sem0, sem1)

        def gather(j, buf, sem):
            return pltpu.make_async_copy(table_hbm.at[idx_v.at[j]], buf, sem)

        gather(0, buf0, sem0).start()

        def body(h, carry):
            for p in range(2):
                j = 2 * h + p
                buf, sem = bufs[p], sems[p]
                gather(j, buf, sem).wait()

                @pl.when(j + 1 < row)
                def _():
                    gather(j + 1, bufs[1 - p], sems[1 - p]).start()

                pltpu.sync_copy(buf, out_hbm.at[j, pl.ds(b0, per_tile)])
            return carry

        lax.fori_loop(0, row // 2, body, 0)

    return k(idx_t, embeddings)


def kernel(inputs, embeddings):
    idx_t = jnp.transpose(inputs.astype(jnp.int32))
    out_t = _lookup_t(idx_t, embeddings)
    return jnp.transpose(out_t, (1, 0, 2))


# speedup vs baseline: 1.8668x; 1.2431x over previous
import functools

import jax
import jax.numpy as jnp
from jax import lax
from jax.experimental import pallas as pl
from jax.experimental.pallas import tpu as pltpu
from jax.experimental.pallas import tpu_sc as plsc

D = 128
NUM_CORES = 2
NUM_SUBCORES = 16
NW = NUM_CORES * NUM_SUBCORES


@jax.jit
def _lookup_t(idx_t, embeddings):
    row, nb = idx_t.shape
    per_tile = nb // NW
    assert nb % NW == 0 and row % 2 == 0
    mesh = plsc.VectorSubcoreMesh(core_axis_name="c", subcore_axis_name="s")

    @functools.partial(
        pl.kernel,
        mesh=mesh,
        out_type=jax.ShapeDtypeStruct((row, nb, D), jnp.float32),
        scratch_types=[
            pltpu.VMEM((row, per_tile), jnp.int32),
            pltpu.VMEM((per_tile, D), jnp.float32),
            pltpu.VMEM((per_tile, D), jnp.float32),
            pltpu.VMEM((per_tile, D), jnp.float32),
            pltpu.VMEM((per_tile, D), jnp.float32),
            pltpu.SemaphoreType.DMA,
            pltpu.SemaphoreType.DMA,
            pltpu.SemaphoreType.DMA,
            pltpu.SemaphoreType.DMA,
        ],
    )
    def k(idx_hbm, table_hbm, out_hbm, idx_v, b_0, b_1, b_2, b_3, s_0, s_1, s_2, s_3):
        wid = lax.axis_index("s") * NUM_CORES + lax.axis_index("c")
        b0 = wid * per_tile
        pltpu.sync_copy(idx_hbm.at[:, pl.ds(b0, per_tile)], idx_v)

        bufs = (b_0, b_1, b_2, b_3)
        sems = (s_0, s_1, s_2, s_3)
        nbuf = 4

        def gather(j, p):
            return pltpu.make_async_copy(table_hbm.at[idx_v.at[j]], bufs[p], sems[p])

        for j in range(nbuf - 1):
            gather(j, j).start()

        def step(j, p, fire=True):
            gather(j, p).wait()
            if fire:
                @pl.when(j + nbuf - 1 < row)
                def _():
                    gather(j + nbuf - 1, (p + nbuf - 1) % nbuf).start()

            pltpu.sync_copy(bufs[p], out_hbm.at[j, pl.ds(b0, per_tile)])

        def body(h, carry):
            for p in range(nbuf):
                step(nbuf * h + p, p)
            return carry

        lax.fori_loop(0, row // nbuf, body, 0)
        for j in range(row - row % nbuf, row):
            step(j, j % nbuf, fire=(j + nbuf - 1 < row))

    return k(idx_t, embeddings)


def kernel(inputs, embeddings):
    idx_t = jnp.transpose(inputs.astype(jnp.int32))
    out_t = _lookup_t(idx_t, embeddings)
    return jnp.transpose(out_t, (1, 0, 2))


# speedup vs baseline: 1.8761x; 1.0050x over previous
import functools

import jax
import jax.numpy as jnp
from jax import lax
from jax.experimental import pallas as pl
from jax.experimental.pallas import tpu as pltpu
from jax.experimental.pallas import tpu_sc as plsc

D = 128
NUM_CORES = 2
NUM_SUBCORES = 16
NW = NUM_CORES * NUM_SUBCORES


@jax.jit
def _lookup_t(idx_t, embeddings):
    row, nb = idx_t.shape
    per_tile = nb // NW
    assert nb % NW == 0 and row % 2 == 0
    mesh = plsc.VectorSubcoreMesh(core_axis_name="c", subcore_axis_name="s")

    @functools.partial(
        pl.kernel,
        mesh=mesh,
        out_type=jax.ShapeDtypeStruct((row, nb, D), jnp.float32),
        scratch_types=[
            pltpu.VMEM((row, per_tile), jnp.int32),
            pltpu.VMEM((per_tile, D), jnp.float32),
            pltpu.VMEM((per_tile, D), jnp.float32),
            pltpu.VMEM((per_tile, D), jnp.float32),
            pltpu.VMEM((per_tile, D), jnp.float32),
            pltpu.VMEM((per_tile, D), jnp.float32),
            pltpu.VMEM((per_tile, D), jnp.float32),
            pltpu.SemaphoreType.DMA,
            pltpu.SemaphoreType.DMA,
            pltpu.SemaphoreType.DMA,
            pltpu.SemaphoreType.DMA,
            pltpu.SemaphoreType.DMA,
            pltpu.SemaphoreType.DMA,
        ],
    )
    def k(idx_hbm, table_hbm, out_hbm, idx_v, b_0, b_1, b_2, b_3, b_4, b_5, s_0, s_1, s_2, s_3, s_4, s_5):
        wid = lax.axis_index("s") * NUM_CORES + lax.axis_index("c")
        b0 = wid * per_tile
        pltpu.sync_copy(idx_hbm.at[:, pl.ds(b0, per_tile)], idx_v)

        bufs = (b_0, b_1, b_2, b_3, b_4, b_5)
        sems = (s_0, s_1, s_2, s_3, s_4, s_5)
        nbuf = 6

        def gather(j, p):
            return pltpu.make_async_copy(table_hbm.at[idx_v.at[j]], bufs[p], sems[p])

        for j in range(nbuf - 1):
            gather(j, j).start()

        def step(j, p, fire=True):
            gather(j, p).wait()
            if fire:
                @pl.when(j + nbuf - 1 < row)
                def _():
                    gather(j + nbuf - 1, (p + nbuf - 1) % nbuf).start()

            pltpu.sync_copy(bufs[p], out_hbm.at[j, pl.ds(b0, per_tile)])

        def body(h, carry):
            for p in range(nbuf):
                step(nbuf * h + p, p)
            return carry

        lax.fori_loop(0, row // nbuf, body, 0)
        for j in range(row - row % nbuf, row):
            step(j, j % nbuf, fire=(j + nbuf - 1 < row))

    return k(idx_t, embeddings)


def kernel(inputs, embeddings):
    idx_t = jnp.transpose(inputs.astype(jnp.int32))
    out_t = _lookup_t(idx_t, embeddings)
    return jnp.transpose(out_t, (1, 0, 2))
